# baseline (speedup 1.0000x reference)
import functools

import jax
import jax.numpy as jnp
from jax import lax
from jax.experimental import pallas as pl
from jax.experimental.pallas import tpu as pltpu
from jax.experimental.pallas import tpu_sc as plsc

_B = 16384
_D = 64
_V = 100001
_L = 16

_info = plsc.get_sparse_core_info()
_NC = _info.num_cores
_NS = _info.num_subcores
_NW = _NC * _NS
_DPW = _D // _NW
_OC = 4096
_NOC = _B // _OC
_UNROLL = 4


def _gather_body(idx_hbm, tab_t_hbm, out_t_hbm,
                 idx_v, row_v, out_v, rsem, isem, osem):
    wid = lax.axis_index("s") * _NC + lax.axis_index("c")
    d0 = wid * _DPW

    def row_cp(d):
        return pltpu.make_async_copy(tab_t_hbm.at[d], row_v, rsem)

    def out_cp(d, k):
        return pltpu.make_async_copy(
            out_v.at[k % 2], out_t_hbm.at[d, pl.ds(k * _OC, _OC)], osem)

    row_cp(d0).start()
    icp = pltpu.make_async_copy(idx_hbm, idx_v, isem)
    icp.start()
    icp.wait()
    row_cp(d0).wait()

    for fd in range(_DPW):
        d = d0 + fd
        for k in range(_NOC):
            buf = k % 2
            if k >= 2:
                out_cp(d, k - 2).wait()

            def gbody(j, _):
                for u in range(_UNROLL):
                    o = (j * _UNROLL + u) * _L
                    out_v[buf, pl.ds(o, _L)] = plsc.load_gather(
                        row_v, [idx_v[pl.ds(k * _OC + o, _L)] + 1])
                return _

            lax.fori_loop(0, _OC // (_L * _UNROLL), gbody, 0)
            out_cp(d, k).start()
        if fd + 1 < _DPW:
            row_cp(d + 1).start()
        for k in (_NOC - 2, _NOC - 1):
            out_cp(d, k).wait()
        if fd + 1 < _DPW:
            row_cp(d + 1).wait()


@jax.jit
def kernel(style_idx, embeddings):
    mesh = plsc.VectorSubcoreMesh(core_axis_name="c", subcore_axis_name="s")
    f = functools.partial(
        pl.kernel,
        mesh=mesh,
        out_type=jax.ShapeDtypeStruct((_D, _B), jnp.float32),
        compiler_params=pltpu.CompilerParams(
            needs_layout_passes=False, skip_device_barrier=True),
        scratch_types=[
            pltpu.VMEM((_B,), jnp.int32),
            pltpu.VMEM((_V,), jnp.float32),
            pltpu.VMEM((2, _OC), jnp.float32),
            pltpu.SemaphoreType.DMA,
            pltpu.SemaphoreType.DMA,
            pltpu.SemaphoreType.DMA,
        ],
    )(_gather_body)
    out_t = f(style_idx, embeddings.T)
    return out_t.T
